# baseline (speedup 1.0000x reference)
import jax
import jax.numpy as jnp
from jax.experimental import pallas as pl
from jax.experimental.pallas import tpu as pltpu

_LATENT = 64
_CODES = 1024
_BLOCK = 1024


def _vq_block_kernel(z_ref, w_ref, zq_ref, loss_ref, lane_ref, wsq_ref):
    w = w_ref[...]

    @pl.when(pl.program_id(0) == 0)
    def _make_consts():
        lane_ref[...] = jax.lax.broadcasted_iota(
            jnp.int32, lane_ref.shape, 1).astype(jnp.float32)
        ones = jnp.ones((1, _LATENT), jnp.float32)
        ww = w * w
        wsq_ref[...] = jax.lax.dot_general(
            ones, ww, (((1,), (1,)), ((), ())),
            preferred_element_type=jnp.float32,
            precision=jax.lax.Precision.HIGHEST)

    x = z_ref[...]
    mm = jax.lax.dot_general(x, w, (((1,), (1,)), ((), ())),
                             preferred_element_type=jnp.float32)
    zsq = jnp.sum(x ** 2, axis=1, keepdims=True)
    d = (zsq + wsq_ref[...]) - 2.0 * mm
    m = jnp.min(d, axis=1, keepdims=True)
    lane = lane_ref[...]
    t = (d - m) * 1e18 + lane
    idx = jnp.min(t, axis=1, keepdims=True)
    onehot = (t == idx).astype(jnp.float32)
    zq = jax.lax.dot_general(onehot, w, (((1,), (0,)), ((), ())),
                             preferred_element_type=jnp.float32)
    zq_ref[...] = zq
    diff = zq - x

    @pl.when(pl.program_id(0) == 0)
    def _init():
        loss_ref[...] = jnp.zeros((1, 1), jnp.float32)

    loss_ref[...] += jnp.sum(diff * diff, keepdims=True)

    @pl.when(pl.program_id(0) == pl.num_programs(0) - 1)
    def _finalize():
        n = pl.num_programs(0) * _BLOCK * _LATENT
        loss_ref[...] *= 2.0 / n


@jax.jit
def kernel(z_e, W):
    bsz, seq, spatial, dlat = z_e.shape
    zf = z_e.reshape(-1, dlat)
    rows = zf.shape[0]

    grid = rows // _BLOCK
    zq_flat, loss_sum = pl.pallas_call(
        _vq_block_kernel,
        grid=(grid,),
        in_specs=[
            pl.BlockSpec((_BLOCK, dlat), lambda i: (i, 0)),
            pl.BlockSpec((_CODES, dlat), lambda i: (0, 0)),
        ],
        out_specs=[
            pl.BlockSpec((_BLOCK, dlat), lambda i: (i, 0)),
            pl.BlockSpec((1, 1), lambda i: (0, 0)),
        ],
        out_shape=[
            jax.ShapeDtypeStruct((rows, dlat), jnp.float32),
            jax.ShapeDtypeStruct((1, 1), jnp.float32),
        ],
        scratch_shapes=[pltpu.VMEM((_BLOCK, _CODES), jnp.float32),
                        pltpu.VMEM((1, _CODES), jnp.float32)],
    )(zf, W)

    loss = loss_sum[0, 0]
    z_q = zq_flat.reshape(z_e.shape)
    return (z_e, loss, z_q)


# speedup vs baseline: 1.0505x; 1.0505x over previous
import jax
import jax.numpy as jnp
from jax.experimental import pallas as pl
from jax.experimental.pallas import tpu as pltpu

_LATENT = 64
_CODES = 1024
_BLOCK = 2048


def _vq_block_kernel(z_ref, w_ref, zq_ref, loss_ref, lane_ref, wsq_ref):
    w = w_ref[...]

    @pl.when(pl.program_id(0) == 0)
    def _make_consts():
        lane_ref[...] = jax.lax.broadcasted_iota(
            jnp.int32, lane_ref.shape, 1).astype(jnp.float32)
        ones = jnp.ones((1, _LATENT), jnp.float32)
        ww = w * w
        wsq_ref[...] = jax.lax.dot_general(
            ones, ww, (((1,), (1,)), ((), ())),
            preferred_element_type=jnp.float32,
            precision=jax.lax.Precision.HIGHEST)

    x = z_ref[...]
    mm = jax.lax.dot_general(x, w, (((1,), (1,)), ((), ())),
                             preferred_element_type=jnp.float32)
    zsq = jnp.sum(x ** 2, axis=1, keepdims=True)
    d = (zsq + wsq_ref[...]) - 2.0 * mm
    m = jnp.min(d, axis=1, keepdims=True)
    lane = lane_ref[...]
    t = (d - m) * 1e18 + lane
    idx = jnp.min(t, axis=1, keepdims=True)
    onehot = (t == idx).astype(jnp.float32)
    zq = jax.lax.dot_general(onehot, w, (((1,), (0,)), ((), ())),
                             preferred_element_type=jnp.float32)
    zq_ref[...] = zq
    diff = zq - x

    @pl.when(pl.program_id(0) == 0)
    def _init():
        loss_ref[...] = jnp.zeros((1, 1), jnp.float32)

    loss_ref[...] += jnp.sum(diff * diff, keepdims=True)

    @pl.when(pl.program_id(0) == pl.num_programs(0) - 1)
    def _finalize():
        n = pl.num_programs(0) * _BLOCK * _LATENT
        loss_ref[...] *= 2.0 / n


@jax.jit
def kernel(z_e, W):
    bsz, seq, spatial, dlat = z_e.shape
    zf = z_e.reshape(-1, dlat)
    rows = zf.shape[0]

    grid = rows // _BLOCK
    zq_flat, loss_sum = pl.pallas_call(
        _vq_block_kernel,
        grid=(grid,),
        in_specs=[
            pl.BlockSpec((_BLOCK, dlat), lambda i: (i, 0)),
            pl.BlockSpec((_CODES, dlat), lambda i: (0, 0)),
        ],
        out_specs=[
            pl.BlockSpec((_BLOCK, dlat), lambda i: (i, 0)),
            pl.BlockSpec((1, 1), lambda i: (0, 0)),
        ],
        out_shape=[
            jax.ShapeDtypeStruct((rows, dlat), jnp.float32),
            jax.ShapeDtypeStruct((1, 1), jnp.float32),
        ],
        scratch_shapes=[pltpu.VMEM((_BLOCK, _CODES), jnp.float32),
                        pltpu.VMEM((1, _CODES), jnp.float32)],
    )(zf, W)

    loss = loss_sum[0, 0]
    z_q = zq_flat.reshape(z_e.shape)
    return (z_e, loss, z_q)
